# Initial kernel scaffold; baseline (speedup 1.0000x reference)
import jax
import jax.numpy as jnp
from jax.experimental import pallas as pl


def kernel(num_recv_tokens_per_expert, expert_start_loc, m_indices):
    raise NotImplementedError("write your pallas kernel here")


# speedup vs baseline: 27.8223x; 27.8223x over previous
import functools

import jax
import jax.numpy as jnp
from jax import lax
from jax.experimental import pallas as pl
from jax.experimental.pallas import tpu as pltpu
from jax.experimental.pallas import tpu_sc as plsc

_E = 64
_T = 262144
_L = 16
_NC = 2
_NS = 16
_NW = _NC * _NS
_CPT = _T // _NW
_NVEC = _CPT // _L


def _tec_body(counts_hbm, m_hbm, out_hbm, counts_v, csum_v, buf_v):
    wid = lax.axis_index("s") * _NC + lax.axis_index("c")
    base = wid * _CPT

    pltpu.sync_copy(counts_hbm, counts_v)

    carry = jnp.int32(0)
    for j in range(_E // _L):
        s = plsc.cumsum(counts_v[pl.ds(j * _L, _L)]) + carry
        csum_v[pl.ds(j * _L, _L)] = s
        carry = jnp.max(s)
    total = carry

    @pl.when(total < base + _CPT)
    def _():
        pltpu.sync_copy(m_hbm.at[pl.ds(base, _CPT)], buf_v)

    @plsc.parallel_loop(0, _NVEC, 1, unroll=8)
    def _(i):
        pos = base + i * _L + lax.iota(jnp.int32, _L)
        res = jnp.zeros((_L,), jnp.int32)
        for step in (32, 16, 8, 4, 2, 1):
            vals = plsc.load_gather(csum_v, [res + (step - 1)])
            res = jnp.where(vals <= pos, res + step, res)
        mvec = buf_v[pl.ds(i * _L, _L)]
        buf_v[pl.ds(i * _L, _L)] = jnp.where(pos < total, res, mvec)

    pltpu.sync_copy(buf_v, out_hbm.at[pl.ds(base, _CPT)])


@jax.jit
def _run(counts, m_indices):
    mesh = plsc.VectorSubcoreMesh(core_axis_name="c", subcore_axis_name="s")
    return pl.kernel(
        _tec_body,
        out_type=jax.ShapeDtypeStruct((_T,), jnp.int32),
        mesh=mesh,
        scratch_types=[
            pltpu.VMEM((_E,), jnp.int32),
            pltpu.VMEM((_E,), jnp.int32),
            pltpu.VMEM((_CPT,), jnp.int32),
        ],
        compiler_params=pltpu.CompilerParams(needs_layout_passes=False),
    )(counts, m_indices)


def kernel(num_recv_tokens_per_expert, expert_start_loc, m_indices):
    del expert_start_loc
    return _run(num_recv_tokens_per_expert, m_indices)
